# baseline (speedup 1.0000x reference)
import functools

import jax
import jax.numpy as jnp
from jax import lax
from jax.experimental import pallas as pl
from jax.experimental.pallas import tpu as pltpu
from jax.experimental.pallas import tpu_sc as plsc

N = 10000
F = 128
HID = 128
NTILES = 16
N_PAD = 10240
ROWS_PER_TILE = N_PAD // NTILES
CHUNK = 88
LANES = 16


NBUF = 4


def _sc_agg_body(xh_hbm, src_hbm, dst_hbm, w_hbm, out_hbm, acc,
                 rows0, rows1, rows2, rows3,
                 sv0, sv1, sv2, sv3,
                 md0, md1, md2, md3,
                 dv0, dv1, dv2, dv3,
                 wv0, wv1, wv2, wv3,
                 g0, g1, g2, g3, s0, s1, s2, s3, m0s, m1s, m2s, m3s):
    core = lax.axis_index("c")
    sid = lax.axis_index("s")
    per_tile = src_hbm.shape[0] // NTILES
    nch = per_tile // CHUNK
    rows = (rows0, rows1, rows2, rows3)
    srcv = (sv0, sv1, sv2, sv3)
    mdst = (md0, md1, md2, md3)
    dstv = (dv0, dv1, dv2, dv3)
    wv = (wv0, wv1, wv2, wv3)
    gsem = (g0, g1, g2, g3)
    ssem = (s0, s1, s2, s3)
    msem = (m0s, m1s, m2s, m3s)
    ebase = sid * per_tile

    def fire_m(k, b):
        off = pl.multiple_of(ebase + k * CHUNK, CHUNK)
        pltpu.async_copy(src_hbm.at[pl.ds(off, CHUNK)], srcv[b], msem[b])
        pltpu.async_copy(dst_hbm.at[pl.ds(off, CHUNK)], mdst[b], msem[b])
        pltpu.async_copy(w_hbm.at[pl.ds(off, CHUNK)], wv[b], msem[b])

    def wait_m(b):
        for _ in range(3):
            pltpu.make_async_copy(src_hbm.at[pl.ds(ebase, CHUNK)],
                                  srcv[b], msem[b]).wait()

    def fire_g(k, b):
        del k
        pltpu.async_copy(xh_hbm.at[core].at[srcv[b]], rows[b], gsem[b])

    def wait_g(k, b):
        del k
        pltpu.make_async_copy(xh_hbm.at[core].at[srcv[b]], rows[b], gsem[b]).wait()

    def fire_s(k, b):
        del k
        for j0 in (0, 16, 32, 48, 64, 72):
            sl = pl.ds(j0, LANES)
            dstv[b][sl] = mdst[b][sl]
        pltpu.async_copy(rows[b], acc.at[dstv[b]], ssem[b], add=True)

    def wait_s(k, b):
        del k
        pltpu.make_async_copy(rows[b], acc.at[dstv[b]], ssem[b]).wait()

    def scale(k, b):
        del k
        rb = rows[b]
        wb = wv[b]

        def srow(r8, rvec):
            for dr in range(8):
                r = r8 * 8 + dr
                wvec = plsc.load_gather(wb, [rvec + dr])
                for f2 in range(F // LANES):
                    sl = pl.ds(f2 * LANES, LANES)
                    rb[r, sl] = rb[r, sl] * wvec
            return rvec + 8

        lax.fori_loop(0, CHUNK // 8, srow, jnp.zeros((LANES,), jnp.int32))

    for b in range(NBUF):
        fire_m(b, b)

    def zrow(r, carry):
        for f2 in range(F // LANES):
            rows0[r, pl.ds(f2 * LANES, LANES)] = jnp.zeros((LANES,), jnp.float32)
        return carry

    lax.fori_loop(0, CHUNK, zrow, 0)
    row0 = sid * ROWS_PER_TILE
    for kk in range(ROWS_PER_TILE // CHUNK):
        pltpu.sync_copy(rows0, acc.at[pl.ds(row0 + kk * CHUNK, CHUNK)])
    zrem = ROWS_PER_TILE - (ROWS_PER_TILE // CHUNK) * CHUNK
    if zrem:
        pltpu.sync_copy(rows0.at[pl.ds(0, zrem)],
                        acc.at[pl.ds(row0 + ROWS_PER_TILE - zrem, zrem)])
    plsc.subcore_barrier()
    for b in range(NBUF):
        wait_m(b)
        fire_g(b, b)

    def outer(g, carry):
        for b in range(NBUF):
            k = g * NBUF + b
            wait_g(k, b)
            scale(k, b)
            fire_s(k, b)
            kf = k + NBUF

            @pl.when(kf < nch)
            def _():
                fire_m(kf, b)

            p = (b + NBUF - 1) % NBUF
            kp = k - 1
            kn = kp + NBUF
            if b == 0:
                @pl.when(g > 0)
                def _():
                    wait_s(kp, p)
                    wait_m(p)
                    fire_g(kn, p)
            else:
                wait_s(kp, p)

                @pl.when(kn < nch)
                def _():
                    wait_m(p)
                    fire_g(kn, p)
        return carry

    lax.fori_loop(0, nch // NBUF, outer, 0)
    wait_s(nch - 1, NBUF - 1)

    plsc.subcore_barrier()
    out_off = core * N_PAD + row0
    pltpu.sync_copy(acc.at[pl.ds(row0, ROWS_PER_TILE)],
                    out_hbm.at[pl.ds(out_off, ROWS_PER_TILE)])


NCH = 232


def _build_sc_agg(interpret=False):
    return pl.kernel(
        _sc_agg_body,
        out_type=jax.ShapeDtypeStruct((2 * N_PAD, F), jnp.float32),
        mesh=plsc.VectorSubcoreMesh(core_axis_name="c", subcore_axis_name="s",
                                    num_cores=2, num_subcores=NTILES),
        scratch_types=(
            [pltpu.VMEM_SHARED((N_PAD, F), jnp.float32)]
            + [pltpu.VMEM((CHUNK, F), jnp.float32)] * NBUF
            + [pltpu.VMEM((CHUNK,), jnp.int32)] * NBUF
            + [pltpu.VMEM((CHUNK,), jnp.int32)] * NBUF
            + [pltpu.VMEM((CHUNK,), jnp.int32)] * NBUF
            + [pltpu.VMEM((CHUNK,), jnp.float32)] * NBUF
            + [pltpu.SemaphoreType.DMA] * (3 * NBUF)
        ),
        compiler_params=pltpu.CompilerParams(needs_layout_passes=False, use_tc_tiling_on_sc=False),
        interpret=interpret,
    )


_SC_AGG_CACHE = []


def _sc_agg(*args):
    if not _SC_AGG_CACHE:
        _SC_AGG_CACHE.append(_build_sc_agg())
    return _SC_AGG_CACHE[0](*args)

BLK = 2000
NBLK = N // BLK


def _tc_body(x_ref, ax_ref, h_ref, ah_ref, c_ref, w_ref, b_ref, wp_ref,
             lng_ref, lnb_ref, fcw_ref, fcb_ref,
             hout_ref, cout_ref, pred_ref):
    z = jnp.concatenate(
        [x_ref[...], ax_ref[...], h_ref[...], ah_ref[...]], axis=1)
    pre = jnp.dot(z, w_ref[...], preferred_element_type=jnp.float32)
    c = c_ref[...]
    b = b_ref[...]
    wp = wp_ref[...]
    ig = jax.nn.sigmoid(pre[:, 0:HID] + wp[0:1] * c + b[0:1])
    fg = jax.nn.sigmoid(pre[:, HID:2 * HID] + wp[1:2] * c + b[1:2])
    gg = jnp.tanh(pre[:, 2 * HID:3 * HID] + b[2:3])
    c_new = fg * c + ig * gg
    og = jax.nn.sigmoid(pre[:, 3 * HID:4 * HID] + wp[2:3] * c_new + b[3:4])
    h_new = og * jnp.tanh(c_new)
    out = jax.nn.relu(h_new)
    mu = jnp.mean(out, axis=-1, keepdims=True)
    var = jnp.mean((out - mu) ** 2, axis=-1, keepdims=True)
    normed = (out - mu) * lax.rsqrt(var + 1e-5) * lng_ref[...] + lnb_ref[...]
    p = jnp.sum(normed * fcw_ref[...], axis=-1, keepdims=True) + fcb_ref[0, 0]
    pred_ref[...] = jnp.broadcast_to(jax.nn.sigmoid(p), (BLK, HID))
    hout_ref[...] = h_new
    cout_ref[...] = c_new


def _build_tc(interpret=False):
    bcast = lambda i: (0, 0)
    row_blk = lambda i: (i, 0)
    return pl.pallas_call(
        _tc_body,
        grid=(NBLK,),
        in_specs=[
            pl.BlockSpec((BLK, F), row_blk),
            pl.BlockSpec((BLK, F), row_blk),
            pl.BlockSpec((BLK, HID), row_blk),
            pl.BlockSpec((BLK, HID), row_blk),
            pl.BlockSpec((BLK, HID), row_blk),
            pl.BlockSpec((4 * F, 4 * HID), bcast),
            pl.BlockSpec((4, HID), bcast),
            pl.BlockSpec((3, HID), bcast),
            pl.BlockSpec((1, HID), bcast),
            pl.BlockSpec((1, HID), bcast),
            pl.BlockSpec((1, HID), bcast),
            pl.BlockSpec((1, 1), bcast),
        ],
        out_specs=[
            pl.BlockSpec((BLK, HID), row_blk),
            pl.BlockSpec((BLK, HID), row_blk),
            pl.BlockSpec((BLK, HID), row_blk),
        ],
        out_shape=[
            jax.ShapeDtypeStruct((N, HID), jnp.float32),
            jax.ShapeDtypeStruct((N, HID), jnp.float32),
            jax.ShapeDtypeStruct((N, HID), jnp.float32),
        ],
        interpret=interpret,
    )


_TC = _build_tc()


def kernel(X, edge_index, edge_weight, skip, H, C, Wx0, Wx1, Wh0, Wh1, b,
           w_peep, ln_g, ln_b, fc_w, fc_b):
    del skip
    x = X[0]
    h = H[0]
    c = C[0]
    e = edge_weight.shape[0]
    e_pad = NTILES * NCH * CHUNK
    pad = e_pad - e
    src = jnp.pad(edge_index[0].astype(jnp.int32), (0, pad))
    dst = jnp.pad(edge_index[1].astype(jnp.int32), (0, pad))
    w = jnp.pad(edge_weight.astype(jnp.float32), (0, pad))

    xh = jnp.stack([x, h], axis=0)
    agg = _sc_agg(xh, src, dst, w)
    aggx = agg[:N]
    aggh = agg[N_PAD:N_PAD + N]

    w_all = jnp.concatenate([Wx0, Wx1, Wh0, Wh1], axis=1)
    w_big = jnp.transpose(w_all, (1, 0, 2)).reshape(4 * F, 4 * HID)

    h_new, c_new, pred = _TC(
        x, aggx, h, aggh, c, w_big, b, w_peep,
        ln_g.reshape(1, HID), ln_b.reshape(1, HID),
        fc_w.reshape(1, HID), fc_b.reshape(1, 1))

    return (pred[:, :1], h_new[None], c_new[None])


# speedup vs baseline: 1.3458x; 1.3458x over previous
import functools

import jax
import jax.numpy as jnp
from jax import lax
from jax.experimental import pallas as pl
from jax.experimental.pallas import tpu as pltpu
from jax.experimental.pallas import tpu_sc as plsc

N = 10000
F = 128
HID = 128
NTILES = 16
N_PAD = 10240
ROWS_PER_TILE = N_PAD // NTILES
CHUNK = 120
LANES = 16


NBUF = 3


def _sc_agg_body(xh_hbm, meta_hbm, out_hbm, acc,
                 rows0, rows1, rows2,
                 mb0, mb1, mb2,
                 dv0, dv1, dv2,
                 g0, g1, g2, s0, s1, s2, m0s, m1s, m2s):
    core = lax.axis_index("c")
    sid = lax.axis_index("s")
    nch_total = meta_hbm.shape[0] // (3 * CHUNK)
    nch = nch_total // NTILES
    rows = (rows0, rows1, rows2)
    mbuf = (mb0, mb1, mb2)
    dstv = (dv0, dv1, dv2)
    gsem = (g0, g1, g2)
    ssem = (s0, s1, s2)
    msem = (m0s, m1s, m2s)
    cbase = sid * nch

    def fire_m(k, b):
        off = pl.multiple_of((cbase + k) * 3 * CHUNK, 3 * CHUNK)
        pltpu.async_copy(meta_hbm.at[pl.ds(off, 3 * CHUNK)], mbuf[b], msem[b])

    def wait_m(b):
        pltpu.make_async_copy(meta_hbm.at[pl.ds(0, 3 * CHUNK)],
                              mbuf[b], msem[b]).wait()

    def fire_g(k, b):
        del k
        pltpu.async_copy(
            xh_hbm.at[core].at[mbuf[b].at[pl.ds(0, CHUNK)]], rows[b], gsem[b])

    def wait_g(k, b):
        del k
        pltpu.make_async_copy(
            xh_hbm.at[core].at[mbuf[b].at[pl.ds(0, CHUNK)]], rows[b],
            gsem[b]).wait()

    def fire_s(k, b):
        del k
        for j0 in (0, 16, 32, 48, 64, 80, 96, 104):
            dstv[b][pl.ds(j0, LANES)] = mbuf[b][pl.ds(CHUNK + j0, LANES)]
        pltpu.async_copy(rows[b], acc.at[dstv[b]], ssem[b], add=True)

    def wait_s(k, b):
        del k
        pltpu.make_async_copy(rows[b], acc.at[dstv[b]], ssem[b]).wait()

    def scale(k, b):
        del k
        rb = rows[b]
        mb = mbuf[b]

        def srow(r8, rvec):
            for dr in range(8):
                r = r8 * 8 + dr
                wvec = plsc.bitcast(plsc.load_gather(mb, [rvec + dr]),
                                    jnp.float32)
                for f2 in range(F // LANES):
                    sl = pl.ds(f2 * LANES, LANES)
                    rb[r, sl] = rb[r, sl] * wvec
            return rvec + 8

        lax.fori_loop(0, CHUNK // 8, srow,
                      jnp.full((LANES,), 2 * CHUNK, jnp.int32))

    for b in range(NBUF):
        fire_m(b, b)

    def zrow(r, carry):
        for f2 in range(F // LANES):
            rows0[r, pl.ds(f2 * LANES, LANES)] = jnp.zeros((LANES,), jnp.float32)
        return carry

    lax.fori_loop(0, CHUNK, zrow, 0)
    row0 = sid * ROWS_PER_TILE
    for kk in range(ROWS_PER_TILE // CHUNK):
        pltpu.sync_copy(rows0, acc.at[pl.ds(row0 + kk * CHUNK, CHUNK)])
    zrem = ROWS_PER_TILE - (ROWS_PER_TILE // CHUNK) * CHUNK
    if zrem:
        pltpu.sync_copy(rows0.at[pl.ds(0, zrem)],
                        acc.at[pl.ds(row0 + ROWS_PER_TILE - zrem, zrem)])
    plsc.subcore_barrier()
    for b in range(NBUF):
        wait_m(b)
        fire_g(b, b)

    def outer(g, carry):
        for b in range(NBUF):
            k = g * NBUF + b
            wait_g(k, b)
            scale(k, b)
            fire_s(k, b)
            kf = k + NBUF

            @pl.when(kf < nch)
            def _():
                fire_m(kf, b)

            p = (b + NBUF - 1) % NBUF
            kp = k - 1
            kn = kp + NBUF
            if b == 0:
                @pl.when(g > 0)
                def _():
                    wait_s(kp, p)
                    wait_m(p)
                    fire_g(kn, p)
            else:
                wait_s(kp, p)

                @pl.when(kn < nch)
                def _():
                    wait_m(p)
                    fire_g(kn, p)
        return carry

    lax.fori_loop(0, nch // NBUF, outer, 0)
    wait_s(nch - 1, NBUF - 1)

    plsc.subcore_barrier()
    out_off = core * N_PAD + row0
    pltpu.sync_copy(acc.at[pl.ds(row0, ROWS_PER_TILE)],
                    out_hbm.at[pl.ds(out_off, ROWS_PER_TILE)])


NCH = 168


def _build_sc_agg(interpret=False):
    return pl.kernel(
        _sc_agg_body,
        out_type=jax.ShapeDtypeStruct((2 * N_PAD, F), jnp.float32),
        mesh=plsc.VectorSubcoreMesh(core_axis_name="c", subcore_axis_name="s",
                                    num_cores=2, num_subcores=NTILES),
        scratch_types=(
            [pltpu.VMEM_SHARED((N_PAD, F), jnp.float32)]
            + [pltpu.VMEM((CHUNK, F), jnp.float32)] * NBUF
            + [pltpu.VMEM((3 * CHUNK,), jnp.int32)] * NBUF
            + [pltpu.VMEM((CHUNK,), jnp.int32)] * NBUF
            + [pltpu.SemaphoreType.DMA] * (3 * NBUF)
        ),
        compiler_params=pltpu.CompilerParams(needs_layout_passes=False, use_tc_tiling_on_sc=False),
        interpret=interpret,
    )


_SC_AGG_CACHE = []


def _sc_agg(*args):
    if not _SC_AGG_CACHE:
        _SC_AGG_CACHE.append(_build_sc_agg())
    return _SC_AGG_CACHE[0](*args)

BLK = 2000
NBLK = N // BLK


def _tc_body(x_ref, ax_ref, h_ref, ah_ref, c_ref, w_ref, b_ref, wp_ref,
             lng_ref, lnb_ref, fcw_ref, fcb_ref,
             hout_ref, cout_ref, pred_ref):
    z = jnp.concatenate(
        [x_ref[...], ax_ref[...], h_ref[...], ah_ref[...]], axis=1)
    pre = jnp.dot(z, w_ref[...], preferred_element_type=jnp.float32)
    c = c_ref[...]
    b = b_ref[...]
    wp = wp_ref[...]
    ig = jax.nn.sigmoid(pre[:, 0:HID] + wp[0:1] * c + b[0:1])
    fg = jax.nn.sigmoid(pre[:, HID:2 * HID] + wp[1:2] * c + b[1:2])
    gg = jnp.tanh(pre[:, 2 * HID:3 * HID] + b[2:3])
    c_new = fg * c + ig * gg
    og = jax.nn.sigmoid(pre[:, 3 * HID:4 * HID] + wp[2:3] * c_new + b[3:4])
    h_new = og * jnp.tanh(c_new)
    out = jax.nn.relu(h_new)
    mu = jnp.mean(out, axis=-1, keepdims=True)
    var = jnp.mean((out - mu) ** 2, axis=-1, keepdims=True)
    normed = (out - mu) * lax.rsqrt(var + 1e-5) * lng_ref[...] + lnb_ref[...]
    p = jnp.sum(normed * fcw_ref[...], axis=-1, keepdims=True) + fcb_ref[0, 0]
    pred_ref[...] = jnp.broadcast_to(jax.nn.sigmoid(p), (BLK, HID))
    hout_ref[...] = h_new
    cout_ref[...] = c_new


def _build_tc(interpret=False):
    bcast = lambda i: (0, 0)
    row_blk = lambda i: (i, 0)
    return pl.pallas_call(
        _tc_body,
        grid=(NBLK,),
        in_specs=[
            pl.BlockSpec((BLK, F), row_blk),
            pl.BlockSpec((BLK, F), row_blk),
            pl.BlockSpec((BLK, HID), row_blk),
            pl.BlockSpec((BLK, HID), row_blk),
            pl.BlockSpec((BLK, HID), row_blk),
            pl.BlockSpec((4 * F, 4 * HID), bcast),
            pl.BlockSpec((4, HID), bcast),
            pl.BlockSpec((3, HID), bcast),
            pl.BlockSpec((1, HID), bcast),
            pl.BlockSpec((1, HID), bcast),
            pl.BlockSpec((1, HID), bcast),
            pl.BlockSpec((1, 1), bcast),
        ],
        out_specs=[
            pl.BlockSpec((BLK, HID), row_blk),
            pl.BlockSpec((BLK, HID), row_blk),
            pl.BlockSpec((BLK, HID), row_blk),
        ],
        out_shape=[
            jax.ShapeDtypeStruct((N, HID), jnp.float32),
            jax.ShapeDtypeStruct((N, HID), jnp.float32),
            jax.ShapeDtypeStruct((N, HID), jnp.float32),
        ],
        interpret=interpret,
    )


_TC = _build_tc()


def kernel(X, edge_index, edge_weight, skip, H, C, Wx0, Wx1, Wh0, Wh1, b,
           w_peep, ln_g, ln_b, fc_w, fc_b):
    del skip
    x = X[0]
    h = H[0]
    c = C[0]
    e = edge_weight.shape[0]
    e_pad = NTILES * NCH * CHUNK
    pad = e_pad - e
    src = jnp.pad(edge_index[0].astype(jnp.int32), (0, pad)).reshape(-1, CHUNK)
    dst = jnp.pad(edge_index[1].astype(jnp.int32), (0, pad)).reshape(-1, CHUNK)
    wbits = lax.bitcast_convert_type(
        jnp.pad(edge_weight.astype(jnp.float32), (0, pad)),
        jnp.int32).reshape(-1, CHUNK)
    meta = jnp.stack([src, dst, wbits], axis=1).reshape(-1)

    xh = jnp.stack([x, h], axis=0)
    agg = _sc_agg(xh, meta)
    aggx = agg[:N]
    aggh = agg[N_PAD:N_PAD + N]

    w_all = jnp.concatenate([Wx0, Wx1, Wh0, Wh1], axis=1)
    w_big = jnp.transpose(w_all, (1, 0, 2)).reshape(4 * F, 4 * HID)

    h_new, c_new, pred = _TC(
        x, aggx, h, aggh, c, w_big, b, w_peep,
        ln_g.reshape(1, HID), ln_b.reshape(1, HID),
        fc_w.reshape(1, HID), fc_b.reshape(1, 1))

    return (pred[:, :1], h_new[None], c_new[None])
